# Initial kernel scaffold; baseline (speedup 1.0000x reference)
import jax
import jax.numpy as jnp
from jax.experimental import pallas as pl


def kernel(hidden, buffer, ptr, W_query, b_query, W_output, b_output, W_compress, b_compress):
    raise NotImplementedError("write your pallas kernel here")


# speedup vs baseline: 2.1686x; 2.1686x over previous
import math

import jax
import jax.numpy as jnp
from jax.experimental import pallas as pl
from jax.experimental.pallas import tpu as pltpu
from jax.experimental.pallas import tpu_sc as plsc
from jax._src.pallas import mpmd as _mpmd

B = 1024
N = 1024
D = 128
BB = 16
SCALE = 1.0 / math.sqrt(D)

_NC = 2
_NS = 16
_NW = _NC * _NS
_RW = B // _NW


def _tc_body(hid_ref, buf_ref, wq_ref, bq_ref, wo_ref, bo_ref,
             wc_ref, bc_ref, read_ref, comp_ref, newbuf_ref):
    h = hid_ref[...]
    q = jax.lax.dot_general(h, wq_ref[...], (((1,), (1,)), ((), ())),
                            preferred_element_type=jnp.float32) + bq_ref[...]
    buf = buf_ref[...]
    logits = jax.lax.dot_general(q, buf, (((1,), (2,)), ((0,), (0,))),
                                 preferred_element_type=jnp.float32) * SCALE
    m = jnp.max(logits, axis=1, keepdims=True)
    e = jnp.exp(logits - m)
    attn = e / jnp.sum(e, axis=1, keepdims=True)
    read = jax.lax.dot_general(attn, buf, (((1,), (1,)), ((0,), (0,))),
                               preferred_element_type=jnp.float32)
    read = jax.lax.dot_general(read, wo_ref[...], (((1,), (1,)), ((), ())),
                               preferred_element_type=jnp.float32) + bo_ref[...]
    read_ref[...] = read
    comp_ref[...] = jax.lax.dot_general(h, wc_ref[...], (((1,), (1,)), ((), ())),
                                        preferred_element_type=jnp.float32) + bc_ref[...]
    newbuf_ref[...] = buf


def _sc_scatter_body(nb_in, comp_hbm, idx_hbm, out_hbm, idx_v, rows_v, sem):
    del nb_in
    wid = jax.lax.axis_index("s") * _NC + jax.lax.axis_index("c")
    base = wid * _RW
    pltpu.sync_copy(idx_hbm.at[pl.ds(base, _RW)], idx_v)
    pltpu.sync_copy(comp_hbm.at[pl.ds(base, _RW)], rows_v)
    pltpu.async_copy(rows_v, out_hbm.at[idx_v], sem).wait()


def kernel(hidden, buffer, ptr, W_query, b_query, W_output, b_output,
           W_compress, b_compress, interpret=False):
    grid = (B // BB,)
    read, comp, new_buffer = pl.pallas_call(
        _tc_body,
        grid=grid,
        in_specs=[
            pl.BlockSpec((BB, D), lambda i: (i, 0)),
            pl.BlockSpec((BB, N, D), lambda i: (i, 0, 0)),
            pl.BlockSpec((D, D), lambda i: (0, 0)),
            pl.BlockSpec((1, D), lambda i: (0, 0)),
            pl.BlockSpec((D, D), lambda i: (0, 0)),
            pl.BlockSpec((1, D), lambda i: (0, 0)),
            pl.BlockSpec((D, D), lambda i: (0, 0)),
            pl.BlockSpec((1, D), lambda i: (0, 0)),
        ],
        out_specs=[
            pl.BlockSpec((BB, D), lambda i: (i, 0)),
            pl.BlockSpec((BB, D), lambda i: (i, 0)),
            pl.BlockSpec((BB, N, D), lambda i: (i, 0, 0)),
        ],
        out_shape=[
            jax.ShapeDtypeStruct((B, D), jnp.float32),
            jax.ShapeDtypeStruct((B, D), jnp.float32),
            jax.ShapeDtypeStruct((B, N, D), jnp.float32),
        ],
        compiler_params=pltpu.CompilerParams(
            dimension_semantics=("arbitrary",),
        ),
        interpret=interpret,
    )(hidden, buffer, W_query, b_query.reshape(1, D),
      W_output, b_output.reshape(1, D), W_compress, b_compress.reshape(1, D))

    row_idx = jnp.arange(B, dtype=jnp.int32) * N + ptr
    mesh = plsc.VectorSubcoreMesh(core_axis_name="c", subcore_axis_name="s")
    scatter = _mpmd._mpmd_map(
        [(mesh, _sc_scatter_body)],
        [jax.ShapeDtypeStruct((B * N, D), jnp.float32)],
        input_output_aliases={0: 0},
        scratch_types=[
            pltpu.VMEM((_RW,), jnp.int32),
            pltpu.VMEM((_RW, D), jnp.float32),
            pltpu.SemaphoreType.DMA,
        ],
    )
    (new_buffer,) = scatter(new_buffer.reshape(B * N, D), comp, row_idx)
    new_ptr = (ptr + 1) % N
    return read, new_buffer.reshape(B, N, D), new_ptr


# speedup vs baseline: 2.1705x; 1.0009x over previous
import math

import jax
import jax.numpy as jnp
from jax.experimental import pallas as pl
from jax.experimental.pallas import tpu as pltpu
from jax.experimental.pallas import tpu_sc as plsc
from jax._src.pallas import mpmd as _mpmd

B = 1024
N = 1024
D = 128
BB = 16
SCALE = 1.0 / math.sqrt(D)

_NC = 2
_NS = 16
_NW = _NC * _NS
_RW = B // _NW


def _tc_body(hid_ref, buf_ref, wq_ref, bq_ref, wo_ref, bo_ref,
             wc_ref, bc_ref, read_ref, comp_ref, newbuf_ref):
    h = hid_ref[...]
    q = jax.lax.dot_general(h, wq_ref[...], (((1,), (1,)), ((), ())),
                            preferred_element_type=jnp.float32) + bq_ref[...]
    buf = buf_ref[...]
    logits = jax.lax.dot_general(q, buf, (((1,), (2,)), ((0,), (0,))),
                                 preferred_element_type=jnp.float32) * SCALE
    m = jnp.max(logits, axis=1, keepdims=True)
    e = jnp.exp(logits - m)
    attn = e / jnp.sum(e, axis=1, keepdims=True)
    read = jax.lax.dot_general(attn, buf, (((1,), (1,)), ((0,), (0,))),
                               preferred_element_type=jnp.float32)
    read = jax.lax.dot_general(read, wo_ref[...], (((1,), (1,)), ((), ())),
                               preferred_element_type=jnp.float32) + bo_ref[...]
    read_ref[...] = read
    comp_ref[...] = jax.lax.dot_general(h, wc_ref[...], (((1,), (1,)), ((), ())),
                                        preferred_element_type=jnp.float32) + bc_ref[...]
    newbuf_ref[...] = buf


def _sc_scatter_body(nb_in, comp_hbm, idx_hbm, out_hbm, idx_v, rows_v,
                     sem_i, sem_r):
    del nb_in
    wid = jax.lax.axis_index("s") * _NC + jax.lax.axis_index("c")
    base = wid * _RW
    cp_i = pltpu.async_copy(idx_hbm.at[pl.ds(base, _RW)], idx_v, sem_i)
    cp_r = pltpu.async_copy(comp_hbm.at[pl.ds(base, _RW)], rows_v, sem_r)
    cp_i.wait()
    cp_r.wait()
    pltpu.async_copy(rows_v, out_hbm.at[idx_v], sem_r).wait()


def kernel(hidden, buffer, ptr, W_query, b_query, W_output, b_output,
           W_compress, b_compress, interpret=False):
    grid = (B // BB,)
    read, comp, new_buffer = pl.pallas_call(
        _tc_body,
        grid=grid,
        in_specs=[
            pl.BlockSpec((BB, D), lambda i: (i, 0)),
            pl.BlockSpec((BB, N, D), lambda i: (i, 0, 0)),
            pl.BlockSpec((D, D), lambda i: (0, 0)),
            pl.BlockSpec((1, D), lambda i: (0, 0)),
            pl.BlockSpec((D, D), lambda i: (0, 0)),
            pl.BlockSpec((1, D), lambda i: (0, 0)),
            pl.BlockSpec((D, D), lambda i: (0, 0)),
            pl.BlockSpec((1, D), lambda i: (0, 0)),
        ],
        out_specs=[
            pl.BlockSpec((BB, D), lambda i: (i, 0)),
            pl.BlockSpec((BB, D), lambda i: (i, 0)),
            pl.BlockSpec((BB, N, D), lambda i: (i, 0, 0)),
        ],
        out_shape=[
            jax.ShapeDtypeStruct((B, D), jnp.float32),
            jax.ShapeDtypeStruct((B, D), jnp.float32),
            jax.ShapeDtypeStruct((B, N, D), jnp.float32),
        ],
        compiler_params=pltpu.CompilerParams(
            dimension_semantics=("arbitrary",),
        ),
        interpret=interpret,
    )(hidden, buffer, W_query, b_query.reshape(1, D),
      W_output, b_output.reshape(1, D), W_compress, b_compress.reshape(1, D))

    row_idx = jnp.arange(B, dtype=jnp.int32) * N + ptr
    mesh = plsc.VectorSubcoreMesh(core_axis_name="c", subcore_axis_name="s")
    scatter = _mpmd._mpmd_map(
        [(mesh, _sc_scatter_body)],
        [jax.ShapeDtypeStruct((B * N, D), jnp.float32)],
        input_output_aliases={0: 0},
        scratch_types=[
            pltpu.VMEM((_RW,), jnp.int32),
            pltpu.VMEM((_RW, D), jnp.float32),
            pltpu.SemaphoreType.DMA,
            pltpu.SemaphoreType.DMA,
        ],
    )
    (new_buffer,) = scatter(new_buffer.reshape(B * N, D), comp, row_idx)
    new_ptr = (ptr + 1) % N
    return read, new_buffer.reshape(B, N, D), new_ptr
